# baseline (speedup 1.0000x reference)
import functools

import jax
import jax.numpy as jnp
from jax import lax
from jax.experimental import pallas as pl
from jax.experimental.pallas import tpu as pltpu
from jax.experimental.pallas import tpu_sc as plsc

N = 10000
E = 160000
NP = 10240
NDUMP = 10240
NH = NP // 2
NACC = NH + 128
NC = 2
NS = 16
NPROD = NC * NS
EPT = 5120
EPAD = 163840
NBD = EPAD // (NS * 128)
SB = 128
SBLOG = 7
NSEG = EPT // SB
GS = 4
QS = SB // GS
W = 256
ROWS_PER_TILE = NP // NS
ACC_PER_TILE = NACC // NS
WB_PER_TILE = NH // NS
BN = 1024
GRID = NP // BN

_mesh = plsc.VectorSubcoreMesh(core_axis_name="c", subcore_axis_name="s")


@functools.partial(
    pl.kernel,
    out_type=jax.ShapeDtypeStruct((NC, NP), jnp.float32),
    mesh=_mesh,
    scratch_types=[
        pltpu.VMEM((NBD, 128), jnp.int32),
        pltpu.VMEM((128,), jnp.float32),
        pltpu.VMEM_SHARED((NP,), jnp.float32),
    ],
)
def _deg_kernel(idx_hbm, zeros_hbm, ones_hbm, out_hbm, idx_v, ones_v, deg_sp):
    c = lax.axis_index("c")
    s = lax.axis_index("s")
    pltpu.sync_copy(ones_hbm, ones_v)
    pltpu.sync_copy(zeros_hbm.at[pl.ds(0, ROWS_PER_TILE)],
                    deg_sp.at[pl.ds(s * ROWS_PER_TILE, ROWS_PER_TILE)])
    pltpu.sync_copy(idx_hbm.at[c, s], idx_v)
    plsc.subcore_barrier()

    def body(b, carry):
        pltpu.sync_copy(ones_v, deg_sp.at[idx_v.at[b]], add=True)
        return carry

    lax.fori_loop(0, NBD, body, None)
    plsc.subcore_barrier()

    @pl.when(s == 0)
    def _():
        pltpu.sync_copy(deg_sp, out_hbm.at[c])


@functools.partial(
    pl.kernel,
    out_type=[
        jax.ShapeDtypeStruct((2, NPROD, NSEG, SB), jnp.int32),
        jax.ShapeDtypeStruct((2, NPROD, NSEG, SB), jnp.int32),
        jax.ShapeDtypeStruct((NC, NS, 2, 16), jnp.int32),
    ],
    mesh=_mesh,
    scratch_types=[
        pltpu.VMEM((EPT // 16, 16), jnp.int32),
        pltpu.VMEM((EPT // 16, 16), jnp.int32),
        pltpu.VMEM((2, NSEG, SB), jnp.int32),
        pltpu.VMEM((2, NSEG, SB), jnp.int32),
        pltpu.VMEM((2, 16), jnp.int32),
    ],
    compiler_params=pltpu.CompilerParams(needs_layout_passes=False),
)
def _part_kernel(src_hbm, dst_hbm, fill_hbm, srcl_hbm, dstl_hbm, cnt_hbm,
                 src_v, dst_v, csrc_v, cdst_v, cnt_v):
    c = lax.axis_index("c")
    s = lax.axis_index("s")
    p = c * NS + s
    pltpu.sync_copy(src_hbm.at[c, s], src_v)
    pltpu.sync_copy(dst_hbm.at[c, s], dst_v)
    for side in range(2):
        pltpu.sync_copy(fill_hbm.at[0], csrc_v.at[side])
        pltpu.sync_copy(fill_hbm.at[1], cdst_v.at[side])

    zero16 = jnp.zeros((16,), jnp.int32)
    iota16 = lax.iota(jnp.int32, 16)

    def _psum16(v):
        for k in (1, 2, 4, 8):
            sh = v.at[jnp.maximum(iota16 - k, 0)].get(mode="promise_in_bounds")
            v = v + jnp.where(iota16 >= k, sh, 0)
        return v

    def body(i, offs):
        off_lo, off_hi = offs
        s16 = src_v[i]
        d16 = dst_v[i]
        mlo = d16 < NH
        mhi = d16 >= NH
        dloc = jnp.where(mlo, d16, d16 - NH)
        clo_sum = _psum16(jnp.where(mlo, 1, 0))
        chi_sum = _psum16(jnp.where(mhi, 1, 0))
        pos_lo = off_lo + clo_sum - 1
        pos_hi = off_hi + chi_sum - 1
        rlo, clo = pos_lo >> SBLOG, pos_lo & (SB - 1)
        rhi, chi = pos_hi >> SBLOG, pos_hi & (SB - 1)
        plsc.store_scatter(csrc_v.at[0], [rlo, clo], s16, mask=mlo)
        plsc.store_scatter(cdst_v.at[0], [rlo, clo], dloc, mask=mlo)
        plsc.store_scatter(csrc_v.at[1], [rhi, chi], s16, mask=mhi)
        plsc.store_scatter(cdst_v.at[1], [rhi, chi], dloc, mask=mhi)
        tot = jnp.full((16,), 15, jnp.int32)
        off_lo = off_lo + clo_sum.at[tot].get(mode="promise_in_bounds")
        off_hi = off_hi + chi_sum.at[tot].get(mode="promise_in_bounds")
        return (off_lo, off_hi)

    off_lo, off_hi = lax.fori_loop(0, EPT // 16, body, (zero16, zero16))
    cnt_v[0] = off_lo
    cnt_v[1] = off_hi
    for side in range(2):
        pltpu.sync_copy(csrc_v.at[side], srcl_hbm.at[side, p])
        pltpu.sync_copy(cdst_v.at[side], dstl_hbm.at[side, p])
    pltpu.sync_copy(cnt_v, cnt_hbm.at[c, s])


def _make_agg_kernel(R):
    scratch = [
        pltpu.VMEM((24, SB), jnp.int32),
        pltpu.VMEM((24, SB), jnp.int32),
        pltpu.VMEM((2, 16), jnp.int32),
        pltpu.VMEM((SB, 2, 128), jnp.float32),
        pltpu.VMEM_SHARED((NACC, 2, 128), jnp.float32),
        pltpu.SemaphoreType.DMA,
    ]

    @functools.partial(
        pl.kernel,
        out_type=jax.ShapeDtypeStruct((R, NC, NH, 2, 128), jnp.float32),
        mesh=_mesh,
        scratch_types=scratch,
    )
    def agg(*refs):
        tables = refs[:R]
        srcl_hbm, dstl_hbm, cnt_hbm, zeros_hbm, out_hbm = refs[R:R + 5]
        src_v, dst_v, cnt_v, rows_v, agg_sp, gsem = refs[R + 5:]
        c = lax.axis_index("c")
        s = lax.axis_index("s")
        for r in range(R):
            pltpu.sync_copy(
                zeros_hbm,
                agg_sp.at[pl.ds(s * ACC_PER_TILE, ACC_PER_TILE)])
            plsc.subcore_barrier()
            table = tables[r]
            for k in range(2):
                p = 2 * s + k
                pltpu.sync_copy(cnt_hbm.at[p // NS, lax.rem(p, NS)], cnt_v)
                cnt = jnp.where(c == 0, cnt_v[0][0], cnt_v[1][0])
                nb_all = lax.div(cnt + (SB - 1), SB)
                for off, sz in ((0, 24), (24, 16)):
                  nb = lax.max(0, lax.min(nb_all - off, sz))

                  @pl.when(nb > 0)
                  def _(table=table, nb=nb, p=p, off=off, sz=sz):
                    pltpu.sync_copy(
                        srcl_hbm.at[c, p, pl.ds(off, sz)],
                        src_v.at[pl.ds(0, sz)])
                    pltpu.sync_copy(
                        dstl_hbm.at[c, p, pl.ds(off, sz)],
                        dst_v.at[pl.ds(0, sz)])

                    def body(b, carry, table=table):
                        for q in range(GS):
                            pltpu.async_copy(
                                table.at[src_v.at[b, pl.ds(q * QS, QS)]],
                                rows_v.at[pl.ds(q * QS, QS)], gsem)
                        pltpu.make_async_copy(
                            table.at[src_v.at[b]], rows_v, gsem).wait()
                        pltpu.sync_copy(
                            rows_v, agg_sp.at[dst_v.at[b]], add=True)
                        return carry

                    lax.fori_loop(0, nb, body, None)

            plsc.subcore_barrier()
            pltpu.sync_copy(
                agg_sp.at[pl.ds(s * WB_PER_TILE, WB_PER_TILE)],
                out_hbm.at[r, c, pl.ds(s * WB_PER_TILE, WB_PER_TILE)])
            if r + 1 < R:
                plsc.subcore_barrier()

    return agg


_agg1 = _make_agg_kernel(1)
_agg2 = _make_agg_kernel(2)


def _norm(deg_blk):
    return lax.rsqrt(jnp.maximum(deg_blk, 1.0))


def _pre_body(x_ref, dout_ref, out_ref):
    xs = x_ref[...] * _norm(dout_ref[...])
    out_ref[:, 0, :] = xs[:, :128]
    out_ref[:, 1, :] = xs[:, 128:]


def _mm1_body(agg_ref, din_ref, dout_ref, w_ref, b_ref, out_ref):
    a = agg_ref[...] * _norm(din_ref[...])
    h = jnp.dot(a, w_ref[...], preferred_element_type=jnp.float32) + b_ref[...]
    h = jnp.maximum(h, 0.0) * _norm(dout_ref[...])
    for j in range(2):
        for t in range(2):
            out_ref[j, :, t, :] = h[:, (2 * j + t) * 128:(2 * j + t + 1) * 128]


def _mm2_body(agg_ref, din_ref, wc2_ref, bc2_ref, wl1_ref, bl1_ref, wo_ref,
              bo_ref, out_ref):
    a = jnp.concatenate([agg_ref[0], agg_ref[1]], axis=1)
    a = a * _norm(din_ref[...])
    h = jnp.dot(a, wc2_ref[...], preferred_element_type=jnp.float32)
    h = jnp.maximum(h + bc2_ref[...], 0.0)
    h = jnp.dot(h, wl1_ref[...], preferred_element_type=jnp.float32)
    h = jnp.maximum(h + bl1_ref[...], 0.0)
    out_ref[...] = (jnp.dot(h, wo_ref[...], preferred_element_type=jnp.float32)
                    + bo_ref[...])


def _full(shape):
    return pl.BlockSpec(shape, lambda i: tuple(0 for _ in shape))


_pre_call = pl.pallas_call(
    _pre_body,
    grid=(GRID,),
    in_specs=[
        pl.BlockSpec((BN, 256), lambda i: (i, 0)),
        pl.BlockSpec((BN, 1), lambda i: (i, 0)),
    ],
    out_specs=pl.BlockSpec((BN, 2, 128), lambda i: (i, 0, 0)),
    out_shape=jax.ShapeDtypeStruct((NP, 2, 128), jnp.float32),
)

_mm1_call = pl.pallas_call(
    _mm1_body,
    grid=(GRID,),
    in_specs=[
        pl.BlockSpec((BN, W), lambda i: (i, 0)),
        pl.BlockSpec((BN, 1), lambda i: (i, 0)),
        pl.BlockSpec((BN, 1), lambda i: (i, 0)),
        _full((256, 512)),
        _full((1, 512)),
    ],
    out_specs=pl.BlockSpec((2, BN, 2, 128), lambda i: (0, i, 0, 0)),
    out_shape=jax.ShapeDtypeStruct((2, NP, 2, 128), jnp.float32),
)

_mm2_call = pl.pallas_call(
    _mm2_body,
    grid=(GRID,),
    in_specs=[
        pl.BlockSpec((2, BN, W), lambda i: (0, i, 0)),
        pl.BlockSpec((BN, 1), lambda i: (i, 0)),
        _full((512, 512)),
        _full((1, 512)),
        _full((512, 512)),
        _full((1, 512)),
        _full((512, 128)),
        _full((1, 128)),
    ],
    out_specs=pl.BlockSpec((BN, 128), lambda i: (i, 0)),
    out_shape=jax.ShapeDtypeStruct((NP, 128), jnp.float32),
)


def kernel(x, edge_index, Wc1, bc1, Wc2, bc2, Wl1, bl1, Wo, bo):
    src = edge_index[0].astype(jnp.int32)
    dst = edge_index[1].astype(jnp.int32)
    pad = EPAD - E
    src_g = jnp.concatenate([src, jnp.zeros((pad,), jnp.int32)])
    dst_g = jnp.concatenate([dst, jnp.full((pad,), NDUMP, jnp.int32)])
    dst_p = jnp.concatenate([dst, jnp.full((pad,), N, jnp.int32)])
    src_d = jnp.concatenate([src, jnp.full((pad,), N, jnp.int32)])

    src_part = src_g.reshape(NC, NS, EPT // 16, 16)
    dst_part = dst_g.reshape(NC, NS, EPT // 16, 16)
    deg_idx = jnp.stack([src_d, dst_p]).reshape(NC, NS, NBD, 128)

    zeros_flat = jnp.zeros((ROWS_PER_TILE,), jnp.float32)
    zeros_w = jnp.zeros((ACC_PER_TILE, 2, 128), jnp.float32)
    ones = jnp.ones((128,), jnp.float32)
    fill = jnp.stack([jnp.zeros((EPT,), jnp.int32),
                      jnp.full((EPT,), NH, jnp.int32)]).reshape(2, NSEG, SB)

    degs = _deg_kernel(deg_idx, zeros_flat, ones)
    deg_out = degs[0].reshape(NP, 1)
    deg_in = degs[1].reshape(NP, 1)

    srcl, dstl, cnts = _part_kernel(src_part, dst_part, fill)

    x_pad = jnp.pad(x, ((0, NP - N), (0, 0)))

    table1 = _pre_call(x_pad, deg_out)
    agg1 = _agg1(table1, srcl, dstl, cnts, zeros_w)
    h1s = _mm1_call(agg1.reshape(NP, W), deg_in, deg_out,
                    Wc1, bc1.reshape(1, 512))

    agg2 = _agg2(h1s[0], h1s[1], srcl, dstl, cnts, zeros_w)
    out = _mm2_call(agg2.reshape(2, NP, W), deg_in,
                    Wc2, bc2.reshape(1, 512),
                    Wl1, bl1.reshape(1, 512),
                    Wo, bo.reshape(1, 128))
    return out[:N]
